# Initial kernel scaffold; baseline (speedup 1.0000x reference)
import jax
import jax.numpy as jnp
from jax.experimental import pallas as pl


def kernel(x, edge_index, edge_weight, node_anchor_adj, graph_skip_conn, W0, W1, W2):
    raise NotImplementedError("write your pallas kernel here")


# speedup vs baseline: 2.5183x; 2.5183x over previous
import functools

import jax
import jax.numpy as jnp
from jax import lax
from jax.experimental import pallas as pl
from jax.experimental.pallas import tpu as pltpu
from jax.experimental.pallas import tpu_sc as plsc

_N = 10000
_E = 320000
_NANCH = 128
_RB = 1000
_GRID = _N // _RB
_CH = 128
_EPAD = 327680
_NCHUNK = _EPAD // _CH
_NTILE = 16
_CPT = _NCHUNK // _NTILE
_STG = 32
_NSTG = _CPT // _STG
_ACCN = 10240
_RPT = _ACCN // _NTILE
_ZB = 128


def _prep(adj):
    def body(adj_ref, invcol_ref, invrow_ref):
        i = pl.program_id(0)
        blk = adj_ref[...]
        rs = jnp.sum(blk, axis=1, keepdims=True)
        invrow_ref[...] = 1.0 / jnp.maximum(rs, 1e-12)
        cs = jnp.sum(blk, axis=0, keepdims=True)

        @pl.when(i == 0)
        def _():
            invcol_ref[...] = cs

        @pl.when(i > 0)
        def _():
            invcol_ref[...] = invcol_ref[...] + cs

        @pl.when(i == _GRID - 1)
        def _():
            invcol_ref[...] = 1.0 / jnp.maximum(invcol_ref[...], 1e-12)

    return pl.pallas_call(
        body,
        grid=(_GRID,),
        in_specs=[pl.BlockSpec((_RB, _NANCH), lambda i: (i, 0))],
        out_specs=[pl.BlockSpec((1, _NANCH), lambda i: (0, 0)),
                   pl.BlockSpec((_RB, 1), lambda i: (i, 0))],
        out_shape=[jax.ShapeDtypeStruct((1, _NANCH), jnp.float32),
                   jax.ShapeDtypeStruct((_N, 1), jnp.float32)],
    )(adj)


def _stage_a(inp, W, adj, invcol, pad_to=None):
    K = inp.shape[1]
    H = W.shape[1]
    H2 = H // 2

    def body(inp_ref, w_ref, adj_ref, invcol_ref, *outs):
        i = pl.program_id(0)
        s = jnp.dot(inp_ref[...], w_ref[...], preferred_element_type=jnp.float32)
        if pad_to is None:
            outs[0][...] = s[:, :H2]
            outs[1][...] = s[:, H2:]
        else:
            outs[0][...] = jnp.concatenate(
                [s, jnp.zeros((_RB, pad_to - H), jnp.float32)], axis=1)
        t_ref = outs[-1]
        nn = adj_ref[...] * invcol_ref[...]
        tt = lax.dot_general(nn, s, (((0,), (0,)), ((), ())),
                             preferred_element_type=jnp.float32)

        @pl.when(i == 0)
        def _():
            t_ref[...] = tt

        @pl.when(i > 0)
        def _():
            t_ref[...] = t_ref[...] + tt

    if pad_to is None:
        sup_specs = [pl.BlockSpec((_RB, H2), lambda i: (i, 0))] * 2
        sup_shapes = [jax.ShapeDtypeStruct((_N, H2), jnp.float32)] * 2
    else:
        sup_specs = [pl.BlockSpec((_RB, pad_to), lambda i: (i, 0))]
        sup_shapes = [jax.ShapeDtypeStruct((_N, pad_to), jnp.float32)]
    return pl.pallas_call(
        body,
        grid=(_GRID,),
        in_specs=[pl.BlockSpec((_RB, K), lambda i: (i, 0)),
                  pl.BlockSpec((K, H), lambda i: (0, 0)),
                  pl.BlockSpec((_RB, _NANCH), lambda i: (i, 0)),
                  pl.BlockSpec((1, _NANCH), lambda i: (0, 0))],
        out_specs=sup_specs + [pl.BlockSpec((_NANCH, H), lambda i: (0, 0))],
        out_shape=sup_shapes + [jax.ShapeDtypeStruct((_NANCH, H), jnp.float32)],
    )(inp, W, adj, invcol)


def _stage_b(adj, invrow, t, agg_a, agg_b, sgate, mode):
    H = t.shape[1]
    HA = agg_a.shape[1]

    def body(adj_ref, invrow_ref, t_ref, aa_ref, ab_ref, sg_ref, *outs):
        sgv = sg_ref[0]
        an = adj_ref[...] * invrow_ref[...]
        anchor = jnp.dot(an, t_ref[...], preferred_element_type=jnp.float32)
        if mode == "last":
            agg = (aa_ref[...] + ab_ref[...])[:, :H]
        else:
            agg = jnp.concatenate([aa_ref[...], ab_ref[...]], axis=1)
        comb = (1.0 - sgv) * anchor + sgv * agg
        if mode == "first":
            outs[0][...] = anchor
            outs[1][...] = agg
            outs[2][...] = jnp.maximum(comb, 0.0)
        elif mode == "mid":
            outs[0][...] = jnp.maximum(comb, 0.0)
        else:
            m = jnp.max(comb, axis=1, keepdims=True)
            lse = jnp.log(jnp.sum(jnp.exp(comb - m), axis=1, keepdims=True))
            outs[0][...] = comb - m - lse

    n_out = 3 if mode == "first" else 1
    return pl.pallas_call(
        body,
        grid=(_GRID,),
        in_specs=[pl.BlockSpec((_RB, _NANCH), lambda i: (i, 0)),
                  pl.BlockSpec((_RB, 1), lambda i: (i, 0)),
                  pl.BlockSpec((_NANCH, H), lambda i: (0, 0)),
                  pl.BlockSpec((_RB, HA), lambda i: (i, 0)),
                  pl.BlockSpec((_RB, HA), lambda i: (i, 0)),
                  pl.BlockSpec(memory_space=pltpu.SMEM)],
        out_specs=[pl.BlockSpec((_RB, H), lambda i: (i, 0))] * n_out,
        out_shape=[jax.ShapeDtypeStruct((_N, H), jnp.float32)] * n_out,
    )(adj, invrow, t, agg_a, agg_b, sgate)


def _spmm(sa, sb, rows2, cols2, wts2, nf_mul=None):
    H2 = sa.shape[1]
    nf = H2 // 16
    nfm = nf if nf_mul is None else nf_mul
    split_edges = sb is None
    stg = 40 if split_edges else _STG
    mesh = plsc.VectorSubcoreMesh(core_axis_name="c", subcore_axis_name="s")

    def body(*args):
        if split_edges:
            (sup_a, rows_hbm, cols_hbm, wts_hbm, out_hbm,
             acc, zb, cb, rb, wb, gb, sem) = args
            sup_b = sup_a
        else:
            (sup_a, sup_b, rows_hbm, cols_hbm, wts_hbm, out_hbm,
             acc, zb, cb, rb, wb, gb, sem) = args
        c = lax.axis_index("c")
        s = lax.axis_index("s")

        def zrow(r, carry):
            for f in range(nf):
                zb[r, pl.ds(f * 16, 16)] = jnp.zeros((16,), jnp.float32)
            return carry

        lax.fori_loop(0, _ZB, zrow, 0)
        for kk in range(_RPT // _ZB):
            pltpu.sync_copy(zb, acc.at[pl.ds(s * _RPT + kk * _ZB, _ZB)])
        plsc.subcore_barrier()

        def edge_pass(sup_hbm):
            if split_edges:
                cpw = _NCHUNK // 32
                wid = c * _NTILE + s
            else:
                cpw = _CPT
                wid = s
            nstg = cpw // stg

            def do_stage(st, carry):
                base = wid * cpw + st * stg
                pltpu.sync_copy(rows_hbm.at[pl.ds(base, stg)], rb)
                pltpu.sync_copy(cols_hbm.at[pl.ds(base, stg)], cb)
                pltpu.sync_copy(wts_hbm.at[pl.ds(base, stg)], wb)

                def do_chunk(k, carry2):
                    pltpu.async_copy(sup_hbm.at[cb.at[k]], gb, sem).wait()

                    def do_group(g, carry3):
                        w16 = wb[k, pl.ds(g * 16, 16)]
                        for jj in range(16):
                            ws = w16[jj]
                            j = g * 16 + jj
                            for f in range(nfm):
                                gb[j, pl.ds(f * 16, 16)] = (
                                    gb[j, pl.ds(f * 16, 16)] * ws)
                        return carry3

                    lax.fori_loop(0, _CH // 16, do_group, 0)
                    pltpu.sync_copy(gb, acc.at[rb.at[k]], add=True)
                    return carry2

                lax.fori_loop(0, stg, do_chunk, 0)
                return carry

            lax.fori_loop(0, nstg, do_stage, 0)

        if split_edges:
            edge_pass(sup_a)
        else:
            @pl.when(c == 0)
            def _():
                edge_pass(sup_a)

            @pl.when(c == 1)
            def _():
                edge_pass(sup_b)

        plsc.subcore_barrier()

        @pl.when(s < _NTILE - 1)
        def _():
            pltpu.sync_copy(acc.at[pl.ds(s * _RPT, _RPT)],
                            out_hbm.at[c, pl.ds(s * _RPT, _RPT)])

        @pl.when(s == _NTILE - 1)
        def _():
            last = (_NTILE - 1) * _RPT
            pltpu.sync_copy(acc.at[pl.ds(last, _N - last)],
                            out_hbm.at[c, pl.ds(last, _N - last)])

    call = pl.kernel(
        body,
        out_type=jax.ShapeDtypeStruct((2, _N, H2), jnp.float32),
        mesh=mesh,
        scratch_types=[
            pltpu.VMEM_SHARED((_ACCN, H2), jnp.float32),
            pltpu.VMEM((_ZB, H2), jnp.float32),
            pltpu.VMEM((stg, _CH), jnp.int32),
            pltpu.VMEM((stg, _CH), jnp.int32),
            pltpu.VMEM((stg, _CH), jnp.float32),
            pltpu.VMEM((_CH, H2), jnp.float32),
            pltpu.SemaphoreType.DMA,
        ],
    )
    if split_edges:
        return call(sa, rows2, cols2, wts2)
    return call(sa, sb, rows2, cols2, wts2)


def kernel(x, edge_index, edge_weight, node_anchor_adj, graph_skip_conn, W0, W1, W2):
    npad = _EPAD - _E
    zpad_i = jnp.zeros((npad,), jnp.int32)
    rows2 = jnp.concatenate([edge_index[0], zpad_i]).reshape(_NCHUNK, _CH)
    cols2 = jnp.concatenate([edge_index[1], zpad_i]).reshape(_NCHUNK, _CH)
    wts2 = jnp.concatenate(
        [edge_weight, jnp.zeros((npad,), jnp.float32)]).reshape(_NCHUNK, _CH)
    adj = node_anchor_adj
    sg = graph_skip_conn

    invcol, invrow = _prep(adj)

    sa0, sb0, t0 = _stage_a(x, W0, adj, invcol)
    agg0 = _spmm(sa0, sb0, rows2, cols2, wts2)
    first_vec, init_agg_vec, nv0 = _stage_b(
        adj, invrow, t0, agg0[0], agg0[1], sg, "first")

    sa1, sb1, t1 = _stage_a(nv0, W1, adj, invcol)
    agg1 = _spmm(sa1, sb1, rows2, cols2, wts2)
    (node_vec,) = _stage_b(adj, invrow, t1, agg1[0], agg1[1], sg, "mid")

    sup2, t2 = _stage_a(node_vec, W2, adj, invcol, pad_to=128)
    agg2 = _spmm(sup2, None, rows2, cols2, wts2, nf_mul=64 // 16)
    (output,) = _stage_b(adj, invrow, t2, agg2[0], agg2[1], sg, "last")

    return (first_vec, init_agg_vec, node_vec, output)
